# Initial kernel scaffold; baseline (speedup 1.0000x reference)
import jax
import jax.numpy as jnp
from jax.experimental import pallas as pl


def kernel(x, edge_index, edge_attr, W0, W1, W2, W3, b0, b1, b2, b3):
    raise NotImplementedError("write your pallas kernel here")


# speedup vs baseline: 2.3305x; 2.3305x over previous
import functools

import jax
import jax.numpy as jnp
from jax import lax
from jax.experimental import pallas as pl
from jax.experimental.pallas import tpu as pltpu
from jax.experimental.pallas import tpu_sc as plsc

N = 10000
D = 128
E = 320000
NC = 2
NS = 16
NW = NC * NS
C = 128
G = 16
NG = 5
CPT = G * NG
EPT = CPT * C
EPAD = NW * EPT
RPT = 640
NPAD = NS * RPT


def _make_shift(n_src):
  mesh = plsc.VectorSubcoreMesh(
      core_axis_name="c", subcore_axis_name="s", num_cores=NC, num_subcores=NS)
  out_type = (
      jax.ShapeDtypeStruct((NPAD, D), jnp.float32),
      jax.ShapeDtypeStruct((NPAD, D), jnp.float32),
  )
  scratch = [
      pltpu.VMEM((G, C), jnp.int32),
      pltpu.VMEM((G, C), jnp.int32),
      pltpu.VMEM((G, C), jnp.float32),
      pltpu.VMEM((C, D), jnp.float32),
  ]
  if n_src == 2:
    scratch.append(pltpu.VMEM((C, D), jnp.float32))
  scratch += [
      pltpu.VMEM_SHARED((NPAD, D), jnp.float32),
      pltpu.SemaphoreType.DMA,
      pltpu.SemaphoreType.DMA,
  ]

  @functools.partial(
      pl.kernel, out_type=out_type, mesh=mesh, scratch_types=scratch)
  def shift(*refs):
    if n_src == 2:
      (h0, h1, srch, dsth, attrh, o0, o1,
       srcv, dstv, attrv, g0, g1, acc, sem0, sem1) = refs
    else:
      (h0, srch, dsth, attrh, o0, o1,
       srcv, dstv, attrv, g0, acc, sem0, sem1) = refs
    cid = lax.axis_index("c")
    sid = lax.axis_index("s")
    wid = sid * NC + cid

    zero = jnp.zeros((16,), jnp.float32)

    def zrow(r, _):
      for u in range(D // 16):
        g0[r, pl.ds(u * 16, 16)] = zero
      return 0

    lax.fori_loop(0, C, zrow, 0)
    for t in range(RPT // C):
      pltpu.sync_copy(g0, acc.at[pl.ds(sid * RPT + t * C, C)])
    plsc.subcore_barrier()

    def group(g, _):
      gbase = g * G
      pltpu.sync_copy(srch.at[wid, pl.ds(gbase, G)], srcv)
      pltpu.sync_copy(dsth.at[wid, pl.ds(gbase, G)], dstv)
      pltpu.sync_copy(attrh.at[wid, pl.ds(gbase, G)], attrv)

      def chunk(j, _):
        cp0 = pltpu.async_copy(h0.at[srcv.at[j]], g0, sem0)
        if n_src == 2:
          cp1 = pltpu.async_copy(h1.at[srcv.at[j]], g1, sem1)
        cp0.wait()
        if n_src == 2:
          cp1.wait()

        def row16(rg, _):
          av = attrv[j, pl.ds(rg * 16, 16)]
          for lane in range(16):
            a = av[lane]
            r = rg * 16 + lane
            for u in range(D // 16):
              sl = pl.ds(u * 16, 16)
              if n_src == 2:
                g0[r, sl] = (g0[r, sl] + g1[r, sl]) * a
              else:
                g0[r, sl] = g0[r, sl] * a
          return 0

        lax.fori_loop(0, C // 16, row16, 0)
        pltpu.sync_copy(g0, acc.at[dstv.at[j]], add=True)
        return 0

      lax.fori_loop(0, G, chunk, 0)
      return 0

    lax.fori_loop(0, NG, group, 0)
    plsc.subcore_barrier()

    rows = pl.ds(sid * RPT, RPT)

    @pl.when(cid == 0)
    def _():
      pltpu.sync_copy(acc.at[rows], o0.at[rows])

    @pl.when(cid == 1)
    def _():
      pltpu.sync_copy(acc.at[rows], o1.at[rows])

  return shift


_shift1 = _make_shift(1)
_shift2 = _make_shift(2)

_BM = 1000


def _mm_body(x_ref, a1, b1, a2, b2, a3, b3, w0, w1, w2, w3, bs, o_ref):
  acc = jnp.dot(x_ref[...], w0[...], preferred_element_type=jnp.float32)
  acc += jnp.dot(a1[...] + b1[...], w1[...], preferred_element_type=jnp.float32)
  acc += jnp.dot(a2[...] + b2[...], w2[...], preferred_element_type=jnp.float32)
  acc += jnp.dot(a3[...] + b3[...], w3[...], preferred_element_type=jnp.float32)
  o_ref[...] = acc + bs[0:1, :]


def _taps_matmul(x, h1a, h1b, h2a, h2b, h3a, h3b, W0, W1, W2, W3, bsum):
  hspec = pl.BlockSpec((_BM, D), lambda i: (i, 0))
  wspec = pl.BlockSpec((D, D), lambda i: (0, 0))
  bspec = pl.BlockSpec((8, D), lambda i: (0, 0))
  return pl.pallas_call(
      _mm_body,
      grid=(N // _BM,),
      in_specs=[hspec] * 7 + [wspec] * 4 + [bspec],
      out_specs=hspec,
      out_shape=jax.ShapeDtypeStruct((N, D), jnp.float32),
  )(x, h1a, h1b, h2a, h2b, h3a, h3b, W0, W1, W2, W3, bsum)


def kernel(x, edge_index, edge_attr, W0, W1, W2, W3, b0, b1, b2, b3):
  src = edge_index[0]
  dst = edge_index[1]
  pad = EPAD - E
  src = jnp.concatenate([src, jnp.zeros((pad,), jnp.int32)]).reshape(NW, CPT, C)
  dst = jnp.concatenate([dst, jnp.zeros((pad,), jnp.int32)]).reshape(NW, CPT, C)
  attr = jnp.concatenate(
      [edge_attr, jnp.zeros((pad,), jnp.float32)]).reshape(NW, CPT, C)

  h1a, h1b = _shift1(x, src, dst, attr)
  h2a, h2b = _shift2(h1a, h1b, src, dst, attr)
  h3a, h3b = _shift2(h2a, h2b, src, dst, attr)

  bsum = jnp.broadcast_to((b0 + b1 + b2 + b3)[None, :], (8, D))
  return _taps_matmul(x, h1a, h1b, h2a, h2b, h3a, h3b, W0, W1, W2, W3, bsum)


# speedup vs baseline: 6.4242x; 2.7565x over previous
import functools

import jax
import jax.numpy as jnp
from jax import lax
from jax.experimental import pallas as pl
from jax.experimental.pallas import tpu as pltpu
from jax.experimental.pallas import tpu_sc as plsc

N = 10000
D = 128
E = 320000
NC = 2
NS = 16
NW = NC * NS
C = 96
G = 21
NG = 5
CPT = G * NG
EPT = CPT * C
EPAD = NW * EPT
RPT = 632
NPAD = NS * RPT
NBUF = 3


def _make_shift():
  mesh = plsc.VectorSubcoreMesh(
      core_axis_name="c", subcore_axis_name="s", num_cores=NC, num_subcores=NS)
  out_type = (
      jax.ShapeDtypeStruct((NPAD, D), jnp.float32),
      jax.ShapeDtypeStruct((NPAD, D), jnp.float32),
  )
  scratch = (
      [
          pltpu.VMEM((G, C), jnp.int32),
          pltpu.VMEM((G, C), jnp.int32),
          pltpu.VMEM((G * C + 16,), jnp.float32),
      ]
      + [pltpu.VMEM((C, D), jnp.float32)] * NBUF
      + [pltpu.VMEM_SHARED((NPAD, D), jnp.float32)]
      + [pltpu.SemaphoreType.DMA] * (2 * NBUF)
  )

  @functools.partial(
      pl.kernel, out_type=out_type, mesh=mesh, scratch_types=scratch)
  def shift(h0, srch, dsth, attrh, o0, o1, srcv, dstv, attrv, *bufs_sems):
    gbuf = bufs_sems[:NBUF]
    acc = bufs_sems[NBUF]
    gsem = bufs_sems[NBUF + 1:NBUF + 1 + NBUF]
    ssem = bufs_sems[NBUF + 1 + NBUF:]
    cid = lax.axis_index("c")
    sid = lax.axis_index("s")
    wid = sid * NC + cid

    zero = jnp.zeros((16,), jnp.float32)

    def zrow(r, _):
      for u in range(D // 16):
        gbuf[0][r, pl.ds(u * 16, 16)] = zero
      return 0

    lax.fori_loop(0, C, zrow, 0)
    base = sid * RPT
    nfull, rem = divmod(RPT, C)
    for t in range(nfull):
      pltpu.sync_copy(gbuf[0], acc.at[pl.ds(base + t * C, C)])
    if rem:
      pltpu.sync_copy(gbuf[0].at[pl.ds(0, rem)],
                      acc.at[pl.ds(base + nfull * C, rem)])
    plsc.subcore_barrier()

    def scale(j, b):
      def row(r, _):
        a = attrv[pl.ds(j * C + r, 16)][0]
        for u in range(D // 16):
          sl = pl.ds(u * 16, 16)
          gbuf[b][r, sl] = gbuf[b][r, sl] * a
        return 0

      lax.fori_loop(0, C, row, 0)

    def group(g, _):
      pltpu.sync_copy(srch.at[wid, g], srcv)
      pltpu.sync_copy(dsth.at[wid, g], dstv)
      pltpu.sync_copy(attrh.at[wid, g], attrv)

      live_g = {}
      live_s = {}
      for j in range(min(2, G)):
        live_g[j % NBUF] = pltpu.async_copy(
            h0.at[srcv.at[j]], gbuf[j % NBUF], gsem[j % NBUF])
      for j in range(G):
        b = j % NBUF
        live_g.pop(b).wait()
        scale(j, b)
        live_s[b] = pltpu.async_copy(
            gbuf[b], acc.at[dstv.at[j]], ssem[b], add=True)
        if j + 2 < G:
          nb = (j + 2) % NBUF
          if nb in live_s:
            live_s.pop(nb).wait()
          live_g[nb] = pltpu.async_copy(
              h0.at[srcv.at[j + 2]], gbuf[nb], gsem[nb])
      for b in sorted(live_s):
        live_s[b].wait()
      return 0

    lax.fori_loop(0, NG, group, 0)
    plsc.subcore_barrier()

    rows = pl.ds(base, RPT)

    @pl.when(cid == 0)
    def _():
      pltpu.sync_copy(acc.at[rows], o0.at[rows])

    @pl.when(cid == 1)
    def _():
      pltpu.sync_copy(acc.at[rows], o1.at[rows])

  return shift


_shift = _make_shift()

_BM = 1000


def _mm_body(x_ref, a1, a2, a3, b3, w0, w1, w2, w3, bs, o_ref):
  acc = jnp.dot(x_ref[...], w0[...], preferred_element_type=jnp.float32)
  acc += jnp.dot(a1[...], w1[...], preferred_element_type=jnp.float32)
  acc += jnp.dot(a2[...], w2[...], preferred_element_type=jnp.float32)
  acc += jnp.dot(a3[...] + b3[...], w3[...], preferred_element_type=jnp.float32)
  o_ref[...] = acc + bs[0:1, :]


def _taps_matmul(x, h1, h2, h3a, h3b, W0, W1, W2, W3, bsum):
  hspec = pl.BlockSpec((_BM, D), lambda i: (i, 0))
  wspec = pl.BlockSpec((D, D), lambda i: (0, 0))
  bspec = pl.BlockSpec((8, D), lambda i: (0, 0))
  return pl.pallas_call(
      _mm_body,
      grid=(N // _BM,),
      in_specs=[hspec] * 5 + [wspec] * 4 + [bspec],
      out_specs=hspec,
      out_shape=jax.ShapeDtypeStruct((N, D), jnp.float32),
  )(x, h1, h2, h3a, h3b, W0, W1, W2, W3, bsum)


def _merge_body(a_ref, b_ref, o_ref):
  o_ref[...] = a_ref[...] + b_ref[...]


def _merge(a, b):
  bm = NPAD // 8
  spec = pl.BlockSpec((bm, D), lambda i: (i, 0))
  return pl.pallas_call(
      _merge_body,
      grid=(NPAD // bm,),
      in_specs=[spec, spec],
      out_specs=spec,
      out_shape=jax.ShapeDtypeStruct((NPAD, D), jnp.float32),
  )(a, b)


def kernel(x, edge_index, edge_attr, W0, W1, W2, W3, b0, b1, b2, b3):
  src = edge_index[0]
  dst = edge_index[1]
  pad = EPAD - E
  shp = (NW, NG, G, C)
  src = jnp.concatenate([src, jnp.zeros((pad,), jnp.int32)]).reshape(shp)
  dst = jnp.concatenate([dst, jnp.zeros((pad,), jnp.int32)]).reshape(shp)
  attr = jnp.concatenate(
      [edge_attr, jnp.zeros((pad,), jnp.float32)]).reshape(NW, NG, G * C)
  attr = jnp.pad(attr, ((0, 0), (0, 0), (0, 16)))

  h1a, h1b = _shift(x, src, dst, attr)
  h1 = _merge(h1a, h1b)
  h2a, h2b = _shift(h1, src, dst, attr)
  h2 = _merge(h2a, h2b)
  h3a, h3b = _shift(h2, src, dst, attr)

  bsum = jnp.broadcast_to((b0 + b1 + b2 + b3)[None, :], (8, D))
  return _taps_matmul(x, h1, h2, h3a, h3b, W0, W1, W2, W3, bsum)
